# Initial kernel scaffold; baseline (speedup 1.0000x reference)
import jax
import jax.numpy as jnp
from jax.experimental import pallas as pl


def kernel(user_emb, item_emb, edge_index, edge_weight, mask, W_attr, b_attr):
    raise NotImplementedError("write your pallas kernel here")


# speedup vs baseline: 1.9886x; 1.9886x over previous
import functools

import jax
import jax.numpy as jnp
from jax import lax
from jax.experimental import pallas as pl
from jax.experimental.pallas import tpu as pltpu
from jax.experimental.pallas import tpu_sc as plsc

NU = 25000
NI = 25000
NN = NU + NI
D = 64
E = 800000
NL = 3

NS = 16
CH = 128
CPT = 391
EPAD = NS * CPT * CH
HALF = NN // 2
PTR = 1568
ACC = NS * PTR
TRASH = 25008
ZR = 112


def _propagate_layer(emb, src, dst, w):
    mesh = plsc.VectorSubcoreMesh(core_axis_name="c", subcore_axis_name="s")

    @functools.partial(
        pl.kernel,
        out_type=jax.ShapeDtypeStruct((NN, D), jnp.float32),
        mesh=mesh,
        scratch_types=[
            pltpu.VMEM((CH,), jnp.int32),
            pltpu.VMEM((CH,), jnp.int32),
            pltpu.VMEM((CH,), jnp.float32),
            pltpu.VMEM((CH, D), jnp.float32),
            pltpu.VMEM((ZR, D), jnp.float32),
            pltpu.VMEM_SHARED((ACC, D), jnp.float32),
            pltpu.SemaphoreType.DMA,
        ],
        compiler_params=pltpu.CompilerParams(use_tc_tiling_on_sc=False),
    )
    def layer(emb_hbm, src_hbm, dst_hbm, w_hbm, out_hbm,
              src_v, dloc_v, w_v, rows_v, zbuf, acc_sh, sem):
        c = lax.axis_index("c")
        s = lax.axis_index("s")

        zv = jnp.zeros((16,), jnp.float32)

        def zrow(r, carry):
            for k in range(D // 16):
                zbuf[r, pl.ds(k * 16, 16)] = zv
            return carry

        lax.fori_loop(0, ZR, zrow, 0)

        def zcopy(j, carry):
            pltpu.sync_copy(zbuf, acc_sh.at[pl.ds(s * PTR + j * ZR, ZR)])
            return carry

        lax.fori_loop(0, PTR // ZR, zcopy, 0)
        plsc.subcore_barrier()

        lo = c * HALF

        def chunk(i, carry):
            base = (s * CPT + i) * CH
            pltpu.sync_copy(src_hbm.at[pl.ds(base, CH)], src_v)
            pltpu.sync_copy(dst_hbm.at[pl.ds(base, CH)], dloc_v)
            pltpu.sync_copy(w_hbm.at[pl.ds(base, CH)], w_v)
            pltpu.async_copy(emb_hbm.at[src_v], rows_v, sem).wait()

            for g in range(CH // 16):
                d16 = dloc_v[pl.ds(g * 16, 16)]
                local = d16 - lo
                ok = (local >= 0) & (local < HALF)
                dloc_v[pl.ds(g * 16, 16)] = jnp.where(ok, local, TRASH)

            def scale(g, carry2):
                w16 = w_v[pl.ds(g * 16, 16)]
                for i in range(16):
                    wv = jnp.full((16,), w16[i], jnp.float32)
                    e = g * 16 + i
                    for k in range(D // 16):
                        rows_v[e, pl.ds(k * 16, 16)] = (
                            rows_v[e, pl.ds(k * 16, 16)] * wv)
                return carry2

            lax.fori_loop(0, CH // 16, scale, 0)

            pltpu.sync_copy(rows_v, acc_sh.at[dloc_v], add=True)
            return carry

        lax.fori_loop(0, CPT, chunk, 0)
        plsc.subcore_barrier()

        start = jnp.minimum(s * PTR, HALF - PTR)
        pltpu.sync_copy(acc_sh.at[pl.ds(start, PTR)],
                        out_hbm.at[pl.ds(lo + start, PTR)])

    return layer(emb, src, dst, w)


def _combine_layers(e0, e1, e2, e3):
    BR = 1000

    def body(a, b, c, d, o):
        o[...] = (a[...] + b[...] + c[...] + d[...]) * 0.25

    bs = pl.BlockSpec((BR, D), lambda i: (i, 0))
    return pl.pallas_call(
        body,
        grid=(NN // BR,),
        in_specs=[bs, bs, bs, bs],
        out_specs=bs,
        out_shape=jax.ShapeDtypeStruct((NN, D), jnp.float32),
    )(e0, e1, e2, e3)


def _user_tail(user_final, mask, W, b2):
    BR = 1000

    def body(u, m, wr, br, mu_ref, pr_ref):
        mu = u[...] * m[...]
        mu_ref[...] = mu
        acc = jnp.dot(mu, wr[...], preferred_element_type=jnp.float32)
        pr_ref[...] = jnp.maximum(acc + br[...], 0.0)

    bs = pl.BlockSpec((BR, D), lambda i: (i, 0))
    bw = pl.BlockSpec((D, D), lambda i: (0, 0))
    bb = pl.BlockSpec((1, D), lambda i: (0, 0))
    return pl.pallas_call(
        body,
        grid=(NU // BR,),
        in_specs=[bs, bs, bw, bb],
        out_specs=[bs, bs],
        out_shape=[jax.ShapeDtypeStruct((NU, D), jnp.float32),
                   jax.ShapeDtypeStruct((NU, D), jnp.float32)],
    )(user_final, mask, W, b2)


def kernel(user_emb, item_emb, edge_index, edge_weight, mask, W_attr, b_attr):
    e0 = jnp.concatenate([user_emb, item_emb], axis=0)
    src = edge_index[0].astype(jnp.int32)
    dst = edge_index[1].astype(jnp.int32)
    pad = EPAD - E
    src = jnp.pad(src, (0, pad))
    dst = jnp.pad(dst, (0, pad))
    w = jnp.pad(edge_weight, (0, pad))

    embs = [e0]
    cur = e0
    for _ in range(NL):
        cur = _propagate_layer(cur, src, dst, w)
        embs.append(cur)

    combined = _combine_layers(*embs)
    user_final = combined[:NU]
    item_final = combined[NU:]
    masked_user_emb, predicted = _user_tail(
        user_final, mask, W_attr, b_attr.reshape(1, D))
    return (user_final, item_final, masked_user_emb, predicted, mask)


# speedup vs baseline: 6.4368x; 3.2369x over previous
import functools

import jax
import jax.numpy as jnp
from jax import lax
from jax.experimental import pallas as pl
from jax.experimental.pallas import tpu as pltpu
from jax.experimental.pallas import tpu_sc as plsc

NU = 25000
NI = 25000
NN = NU + NI
D = 64
DH = D // 2
E = 800000
NL = 3

NS = 16
CH = 128
CPS = 16
NSC = 25
EPT = NSC * CPS * CH
EPAD = NS * EPT
EROWS = EPAD // CH
RPT = EPT // CH

PTR = 3136
ACC = NS * PTR
ZR = 392


def _propagate_layer(emb_flat, se2, de2, we2, mul, off):
    mesh = plsc.VectorSubcoreMesh(core_axis_name="c", subcore_axis_name="s")

    @functools.partial(
        pl.kernel,
        out_type=jax.ShapeDtypeStruct((2, NN, DH), jnp.float32),
        mesh=mesh,
        scratch_types=[
            pltpu.VMEM((CPS, CH), jnp.int32),
            pltpu.VMEM((CPS, CH), jnp.int32),
            pltpu.VMEM((CPS, CH), jnp.float32),
            pltpu.VMEM((3, CH, DH), jnp.float32),
            pltpu.VMEM((2, CH, DH), jnp.float32),
            pltpu.VMEM_SHARED((ACC, DH), jnp.float32),
            pltpu.SemaphoreType.DMA((3,)),
            pltpu.SemaphoreType.DMA((2,)),
        ],
        compiler_params=pltpu.CompilerParams(use_tc_tiling_on_sc=False),
    )
    def layer(emb_hbm, se_hbm, de_hbm, we_hbm, out_hbm,
              seb, deb, web, gb, sb, acc, gsem, ssem):
        c = lax.axis_index("c")
        s = lax.axis_index("s")
        roff = c * off

        zv = jnp.zeros((16,), jnp.float32)

        def zrow(r, carry):
            for k in range(DH // 16):
                gb[0, r, pl.ds(k * 16, 16)] = zv
            return carry

        lax.fori_loop(0, CH, zrow, 0)

        def zcopy(j, carry):
            pltpu.sync_copy(gb.at[0], acc.at[pl.ds(s * PTR + j * CH, CH)])
            return carry

        lax.fori_loop(0, PTR // CH, zcopy, 0)
        pltpu.sync_copy(gb.at[0, pl.ds(0, PTR - (PTR // CH) * CH)],
                        acc.at[pl.ds(s * PTR + (PTR // CH) * CH,
                                     PTR - (PTR // CH) * CH)])
        plsc.subcore_barrier()

        def superchunk(sc, carry):
            rb = s * RPT + sc * CPS
            pltpu.sync_copy(se_hbm.at[pl.ds(rb, CPS)], seb)
            pltpu.sync_copy(de_hbm.at[pl.ds(rb, CPS)], deb)
            pltpu.sync_copy(we_hbm.at[pl.ds(rb, CPS)], web)

            def gixrow(j, carry2):
                def gix(g, carry3):
                    sl = pl.ds(g * 16, 16)
                    seb[j, sl] = seb[j, sl] * mul + roff
                    return carry3
                return lax.fori_loop(0, CH // 16, gix, carry2)

            lax.fori_loop(0, CPS, gixrow, 0)

            def gather(j):
                return pltpu.async_copy(
                    emb_hbm.at[seb.at[j]], gb.at[j % 3], gsem.at[j % 3])

            gathers = {j: gather(j) for j in range(3)}
            scatters = {}
            for j in range(CPS):
                bg, bs = j % 3, j % 2
                gathers[j].wait()
                if j >= 2:
                    scatters[j - 2].wait()

                def scale(g, carry2, j=j, bg=bg, bs=bs):
                    w16 = web[j, pl.ds(g * 16, 16)]
                    for i in range(16):
                        wv = jnp.full((16,), w16[i], jnp.float32)
                        e = g * 16 + i
                        for k in range(DH // 16):
                            sb[bs, e, pl.ds(k * 16, 16)] = (
                                gb[bg, e, pl.ds(k * 16, 16)] * wv)
                    return carry2

                lax.fori_loop(0, CH // 16, scale, 0)
                scatters[j] = pltpu.async_copy(
                    sb.at[bs], acc.at[deb.at[j]], ssem.at[bs], add=True)
                if j + 3 < CPS:
                    gathers[j + 3] = gather(j + 3)
            scatters[CPS - 2].wait()
            scatters[CPS - 1].wait()
            return carry

        lax.fori_loop(0, NSC, superchunk, 0)
        plsc.subcore_barrier()

        start = jnp.minimum(s * PTR, NN - PTR)
        pltpu.sync_copy(acc.at[pl.ds(start, PTR)],
                        out_hbm.at[c].at[pl.ds(start, PTR)])

    return layer(emb_flat, se2, de2, we2)


BR = 1000


def _concat_emb(user_emb, item_emb):
    nb = NU // BR

    def body(u, it, o):
        i = pl.program_id(0)

        @pl.when(i < nb)
        def _():
            o[...] = u[...]

        @pl.when(i >= nb)
        def _():
            o[...] = it[...]

    return pl.pallas_call(
        body,
        grid=(NN // BR,),
        in_specs=[pl.BlockSpec((BR, D), lambda i: (jnp.minimum(i, nb - 1), 0)),
                  pl.BlockSpec((BR, D), lambda i: (jnp.maximum(i - nb, 0), 0))],
        out_specs=pl.BlockSpec((BR, D), lambda i: (i, 0)),
        out_shape=jax.ShapeDtypeStruct((NN, D), jnp.float32),
    )(user_emb, item_emb)


def _half_specs(offset_blocks):
    specs = []
    for h in range(2):
        specs.append(pl.BlockSpec(
            (1, BR, DH), lambda i, h=h: (h, i + offset_blocks, 0)))
    return specs


def _user_tail(e0, e1, e2, e3, mask, W, b2):
    def body(e0r, aL, aH, bL, bH, cL, cH, mr, wr, br, uf, mu_ref, pr_ref):
        lo = (e0r[:, :DH] + aL[0] + bL[0] + cL[0]) * 0.25
        hi = (e0r[:, DH:] + aH[0] + bH[0] + cH[0]) * 0.25
        comb = jnp.concatenate([lo, hi], axis=1)
        uf[...] = comb
        mu = comb * mr[...]
        mu_ref[...] = mu
        acc = jnp.dot(mu, wr[...], preferred_element_type=jnp.float32)
        pr_ref[...] = jnp.maximum(acc + br[...], 0.0)

    bs = pl.BlockSpec((BR, D), lambda i: (i, 0))
    bw = pl.BlockSpec((D, D), lambda i: (0, 0))
    bb = pl.BlockSpec((1, D), lambda i: (0, 0))
    return pl.pallas_call(
        body,
        grid=(NU // BR,),
        in_specs=([bs] + _half_specs(0) + _half_specs(0) + _half_specs(0)
                  + [bs, bw, bb]),
        out_specs=[bs, bs, bs],
        out_shape=[jax.ShapeDtypeStruct((NU, D), jnp.float32)] * 3,
    )(e0, e1, e1, e2, e2, e3, e3, mask, W, b2)


def _item_tail(e0, e1, e2, e3):
    off = NU // BR

    def body(e0r, aL, aH, bL, bH, cL, cH, it):
        lo = (e0r[:, :DH] + aL[0] + bL[0] + cL[0]) * 0.25
        hi = (e0r[:, DH:] + aH[0] + bH[0] + cH[0]) * 0.25
        it[...] = jnp.concatenate([lo, hi], axis=1)

    bs = pl.BlockSpec((BR, D), lambda i: (i + off, 0))
    return pl.pallas_call(
        body,
        grid=(NI // BR,),
        in_specs=[bs] + _half_specs(off) + _half_specs(off) + _half_specs(off),
        out_specs=pl.BlockSpec((BR, D), lambda i: (i, 0)),
        out_shape=jax.ShapeDtypeStruct((NI, D), jnp.float32),
    )(e0, e1, e1, e2, e2, e3, e3)


def kernel(user_emb, item_emb, edge_index, edge_weight, mask, W_attr, b_attr):
    e0 = _concat_emb(user_emb, item_emb)
    pad = EPAD - E
    src = jnp.pad(edge_index[0].astype(jnp.int32), (0, pad)).reshape(EROWS, CH)
    dst = jnp.pad(edge_index[1].astype(jnp.int32), (0, pad)).reshape(EROWS, CH)
    w = jnp.pad(edge_weight, (0, pad)).reshape(EROWS, CH)

    cur = _propagate_layer(e0.reshape(2 * NN, DH), src, dst, w, 2, 1)
    layers = [cur]
    for _ in range(NL - 1):
        cur = _propagate_layer(cur.reshape(2 * NN, DH), src, dst, w, 1, NN)
        layers.append(cur)

    e1, e2, e3 = layers
    user_final, masked_user_emb, predicted = _user_tail(
        e0, e1, e2, e3, mask, W_attr, b_attr.reshape(1, D))
    item_final = _item_tail(e0, e1, e2, e3)
    return (user_final, item_final, masked_user_emb, predicted, mask)


# speedup vs baseline: 6.6361x; 1.0310x over previous
import functools

import jax
import jax.numpy as jnp
from jax import lax
from jax.experimental import pallas as pl
from jax.experimental.pallas import tpu as pltpu
from jax.experimental.pallas import tpu_sc as plsc

NU = 25000
NI = 25000
NN = NU + NI
D = 64
DH = D // 2
E = 800000
NL = 3

NS = 16
CH = 128
CPS = 16
NSC = 25
EPT = NSC * CPS * CH
EPAD = NS * EPT
EROWS = EPAD // CH
RPT = EPT // CH

PTR = 3136
ACC = NS * PTR
ZR = 392


def _shadow0(e0):
    mesh = plsc.VectorSubcoreMesh(core_axis_name="c", subcore_axis_name="s")

    @functools.partial(
        pl.kernel,
        out_type=jax.ShapeDtypeStruct((2, NN, DH), jnp.bfloat16),
        mesh=mesh,
        scratch_types=[
            pltpu.VMEM((CH, D), jnp.float32),
            pltpu.VMEM((CH, DH), jnp.bfloat16),
        ],
        compiler_params=pltpu.CompilerParams(use_tc_tiling_on_sc=False,
                                             needs_layout_passes=False),
    )
    def conv(e0_hbm, sh_hbm, slab, shb):
        c = lax.axis_index("c")
        s = lax.axis_index("s")
        start = jnp.minimum(s * PTR, NN - PTR)
        coff = c * DH

        def do_slab(rstart, size):
            pltpu.sync_copy(e0_hbm.at[pl.ds(rstart, size)],
                            slab.at[pl.ds(0, size)])

            def prow(r, carry):
                a = slab[r, pl.ds(coff, 16)]
                b = slab[r, pl.ds(coff + 16, 16)]
                shb[r, :] = plsc.pack(
                    a, b, format=plsc.PackFormat.INTERLEAVED,
                    preferred_element_type=jnp.bfloat16)
                return carry

            lax.fori_loop(0, size, prow, 0)
            pltpu.sync_copy(shb.at[pl.ds(0, size)],
                            sh_hbm.at[c].at[pl.ds(rstart, size)])

        def slab_loop(t, carry):
            do_slab(start + t * CH, CH)
            return carry

        lax.fori_loop(0, PTR // CH, slab_loop, 0)
        do_slab(start + (PTR // CH) * CH, PTR - (PTR // CH) * CH)

    return conv(e0)


def _propagate_layer(sh_flat, se2, de2, we2, emit_shadow):
    mesh = plsc.VectorSubcoreMesh(core_axis_name="c", subcore_axis_name="s")

    out_f32 = jax.ShapeDtypeStruct((2, NN, DH), jnp.float32)
    out_sh = jax.ShapeDtypeStruct((2, NN, DH), jnp.bfloat16)
    out_type = (out_f32, out_sh) if emit_shadow else out_f32

    @functools.partial(
        pl.kernel,
        out_type=out_type,
        mesh=mesh,
        scratch_types=[
            pltpu.VMEM((CPS, CH), jnp.int32),
            pltpu.VMEM((CPS, CH), jnp.int32),
            pltpu.VMEM((CPS, CH), jnp.float32),
            pltpu.VMEM((3, CH, DH), jnp.bfloat16),
            pltpu.VMEM((2, CH, DH), jnp.float32),
            pltpu.VMEM_SHARED((ACC, DH), jnp.float32),
            pltpu.SemaphoreType.DMA((3,)),
            pltpu.SemaphoreType.DMA((2,)),
        ],
        compiler_params=pltpu.CompilerParams(use_tc_tiling_on_sc=False, needs_layout_passes=False),
    )
    def layer(sh_hbm, se_hbm, de_hbm, we_hbm, *out_and_scratch):
        if emit_shadow:
            (outf_hbm, outsh_hbm,
             seb, deb, web, gb, sb, acc, gsem, ssem) = out_and_scratch
        else:
            (outf_hbm,
             seb, deb, web, gb, sb, acc, gsem, ssem) = out_and_scratch
        c = lax.axis_index("c")
        s = lax.axis_index("s")
        roff = c * NN

        zv = jnp.zeros((16,), jnp.float32)

        def zrow(r, carry):
            for k in range(DH // 16):
                sb[0, r, pl.ds(k * 16, 16)] = zv
            return carry

        lax.fori_loop(0, CH, zrow, 0)

        def zcopy(j, carry):
            pltpu.sync_copy(sb.at[0], acc.at[pl.ds(s * PTR + j * CH, CH)])
            return carry

        lax.fori_loop(0, PTR // CH, zcopy, 0)
        pltpu.sync_copy(sb.at[0, pl.ds(0, PTR - (PTR // CH) * CH)],
                        acc.at[pl.ds(s * PTR + (PTR // CH) * CH,
                                     PTR - (PTR // CH) * CH)])
        plsc.subcore_barrier()

        def superchunk(sc, carry):
            rb = s * RPT + sc * CPS
            pltpu.sync_copy(se_hbm.at[pl.ds(rb, CPS)], seb)
            pltpu.sync_copy(de_hbm.at[pl.ds(rb, CPS)], deb)
            pltpu.sync_copy(we_hbm.at[pl.ds(rb, CPS)], web)

            def gixrow(j, carry2):
                def gix(g, carry3):
                    sl = pl.ds(g * 16, 16)
                    seb[j, sl] = seb[j, sl] + roff
                    return carry3
                return lax.fori_loop(0, CH // 16, gix, carry2)

            lax.fori_loop(0, CPS, gixrow, 0)

            def gather(j):
                return pltpu.async_copy(
                    sh_hbm.at[seb.at[j]], gb.at[j % 3], gsem.at[j % 3])

            gathers = {j: gather(j) for j in range(3)}
            scatters = {}
            for j in range(CPS):
                bg, bs = j % 3, j % 2
                gathers[j].wait()
                if j >= 2:
                    scatters[j - 2].wait()

                def scale(g, carry2, j=j, bg=bg, bs=bs):
                    w16 = web[j, pl.ds(g * 16, 16)]
                    for i in range(16):
                        wv = jnp.full((16,), w16[i], jnp.float32)
                        e = g * 16 + i
                        a, b = plsc.unpack(
                            gb[bg, e, :],
                            format=plsc.PackFormat.INTERLEAVED,
                            preferred_element_type=jnp.float32)
                        sb[bs, e, pl.ds(0, 16)] = a * wv
                        sb[bs, e, pl.ds(16, 16)] = b * wv
                    return carry2

                lax.fori_loop(0, CH // 16, scale, 0)
                scatters[j] = pltpu.async_copy(
                    sb.at[bs], acc.at[deb.at[j]], ssem.at[bs], add=True)
                if j + 3 < CPS:
                    gathers[j + 3] = gather(j + 3)
            scatters[CPS - 2].wait()
            scatters[CPS - 1].wait()
            return carry

        lax.fori_loop(0, NSC, superchunk, 0)
        plsc.subcore_barrier()

        start = jnp.minimum(s * PTR, NN - PTR)
        pltpu.sync_copy(acc.at[pl.ds(start, PTR)],
                        outf_hbm.at[c].at[pl.ds(start, PTR)])

        if emit_shadow:
            def sh_slab(rstart, size):
                pltpu.sync_copy(acc.at[pl.ds(rstart, size)],
                                sb.at[0, pl.ds(0, size)])

                def prow(r, carry):
                    a = sb[0, r, pl.ds(0, 16)]
                    b = sb[0, r, pl.ds(16, 16)]
                    gb[0, r, :] = plsc.pack(
                        a, b, format=plsc.PackFormat.INTERLEAVED,
                        preferred_element_type=jnp.bfloat16)
                    return carry

                lax.fori_loop(0, size, prow, 0)
                pltpu.sync_copy(gb.at[0, pl.ds(0, size)],
                                outsh_hbm.at[c].at[pl.ds(rstart, size)])

            def sh_loop(t, carry):
                sh_slab(start + t * CH, CH)
                return carry

            lax.fori_loop(0, PTR // CH, sh_loop, 0)
            sh_slab(start + (PTR // CH) * CH, PTR - (PTR // CH) * CH)

    return layer(sh_flat, se2, de2, we2)


BR = 1000


def _concat_emb(user_emb, item_emb):
    nb = NU // BR

    def body(u, it, o):
        i = pl.program_id(0)

        @pl.when(i < nb)
        def _():
            o[...] = u[...]

        @pl.when(i >= nb)
        def _():
            o[...] = it[...]

    return pl.pallas_call(
        body,
        grid=(NN // BR,),
        in_specs=[pl.BlockSpec((BR, D), lambda i: (jnp.minimum(i, nb - 1), 0)),
                  pl.BlockSpec((BR, D), lambda i: (jnp.maximum(i - nb, 0), 0))],
        out_specs=pl.BlockSpec((BR, D), lambda i: (i, 0)),
        out_shape=jax.ShapeDtypeStruct((NN, D), jnp.float32),
    )(user_emb, item_emb)


def _half_specs(offset_blocks):
    specs = []
    for h in range(2):
        specs.append(pl.BlockSpec(
            (1, BR, DH), lambda i, h=h: (h, i + offset_blocks, 0)))
    return specs


def _user_tail(e0, e1, e2, e3, mask, W, b2):
    def body(e0r, aL, aH, bL, bH, cL, cH, mr, wr, br, uf, mu_ref, pr_ref):
        lo = (e0r[:, :DH] + aL[0] + bL[0] + cL[0]) * 0.25
        hi = (e0r[:, DH:] + aH[0] + bH[0] + cH[0]) * 0.25
        comb = jnp.concatenate([lo, hi], axis=1)
        uf[...] = comb
        mu = comb * mr[...]
        mu_ref[...] = mu
        acc = jnp.dot(mu, wr[...], preferred_element_type=jnp.float32)
        pr_ref[...] = jnp.maximum(acc + br[...], 0.0)

    bs = pl.BlockSpec((BR, D), lambda i: (i, 0))
    bw = pl.BlockSpec((D, D), lambda i: (0, 0))
    bb = pl.BlockSpec((1, D), lambda i: (0, 0))
    return pl.pallas_call(
        body,
        grid=(NU // BR,),
        in_specs=([bs] + _half_specs(0) + _half_specs(0) + _half_specs(0)
                  + [bs, bw, bb]),
        out_specs=[bs, bs, bs],
        out_shape=[jax.ShapeDtypeStruct((NU, D), jnp.float32)] * 3,
    )(e0, e1, e1, e2, e2, e3, e3, mask, W, b2)


def _item_tail(e0, e1, e2, e3):
    off = NU // BR

    def body(e0r, aL, aH, bL, bH, cL, cH, it):
        lo = (e0r[:, :DH] + aL[0] + bL[0] + cL[0]) * 0.25
        hi = (e0r[:, DH:] + aH[0] + bH[0] + cH[0]) * 0.25
        it[...] = jnp.concatenate([lo, hi], axis=1)

    bs = pl.BlockSpec((BR, D), lambda i: (i + off, 0))
    return pl.pallas_call(
        body,
        grid=(NI // BR,),
        in_specs=[bs] + _half_specs(off) + _half_specs(off) + _half_specs(off),
        out_specs=pl.BlockSpec((BR, D), lambda i: (i, 0)),
        out_shape=jax.ShapeDtypeStruct((NI, D), jnp.float32),
    )(e0, e1, e1, e2, e2, e3, e3)


def kernel(user_emb, item_emb, edge_index, edge_weight, mask, W_attr, b_attr):
    e0 = _concat_emb(user_emb, item_emb)
    pad = EPAD - E
    src = jnp.pad(edge_index[0].astype(jnp.int32), (0, pad)).reshape(EROWS, CH)
    dst = jnp.pad(edge_index[1].astype(jnp.int32), (0, pad)).reshape(EROWS, CH)
    w = jnp.pad(edge_weight, (0, pad)).reshape(EROWS, CH)

    sh = _shadow0(e0)
    e1, sh = _propagate_layer(sh.reshape(2 * NN, DH), src, dst, w, True)
    e2, sh = _propagate_layer(sh.reshape(2 * NN, DH), src, dst, w, True)
    e3 = _propagate_layer(sh.reshape(2 * NN, DH), src, dst, w, False)
    user_final, masked_user_emb, predicted = _user_tail(
        e0, e1, e2, e3, mask, W_attr, b_attr.reshape(1, D))
    item_final = _item_tail(e0, e1, e2, e3)
    return (user_final, item_final, masked_user_emb, predicted, mask)
